# baseline (speedup 1.0000x reference)
import functools

import jax
import jax.numpy as jnp
from jax import lax
from jax.experimental import pallas as pl
from jax.experimental.pallas import tpu as pltpu
from jax.experimental.pallas import tpu_sc as plsc

_N_NODES = 100000
_NUM_SPECIES = 119
_DIM = 128

_info = plsc.get_sparse_core_info()
_NC = _info.num_cores
_NS = _info.num_subcores
_NW = _NC * _NS

_CHUNK = 128
_HALF = _CHUNK // 2
_FULL = _N_NODES // _CHUNK
_TAIL = _N_NODES - _FULL * _CHUNK
_K_LO = _FULL // _NW
_N_HI = _FULL - _K_LO * _NW
_K_HI = _K_LO + 1

_mesh = plsc.VectorSubcoreMesh(core_axis_name="c", subcore_axis_name="s")


@functools.partial(
    pl.kernel,
    mesh=_mesh,
    out_type=jax.ShapeDtypeStruct((_N_NODES, _DIM), jnp.float32),
    scratch_types=[
        pltpu.VMEM((_NUM_SPECIES, _DIM), jnp.float32),
        pltpu.VMEM((_NUM_SPECIES + 17, ), jnp.float32),
        pltpu.VMEM((_K_HI * _CHUNK,), jnp.int32),
        pltpu.VMEM((_HALF, _DIM), jnp.float32),
        pltpu.VMEM((_HALF, _DIM), jnp.float32),
        pltpu.SemaphoreType.DMA,
        pltpu.SemaphoreType.DMA,
        pltpu.SemaphoreType.DMA,
        pltpu.SemaphoreType.DMA,
        pltpu.SemaphoreType.DMA,
    ],
)
def _sc_embed(table_hbm, mass_hbm, idx_hbm, out_hbm,
              tbl_v, mass_v, idx_v, stage_a, stage_b,
              sem_tbl, sem_m, sem_i, sem_oa, sem_ob):
    wid = lax.axis_index("s") * _NC + lax.axis_index("c")
    is_hi = wid < _N_HI
    base = lax.select(is_hi, wid * _K_HI,
                      _N_HI * _K_HI + (wid - _N_HI) * _K_LO)
    row0 = base * _CHUNK
    nk = lax.select(is_hi, _K_HI, _K_LO)

    ct = pltpu.async_copy(table_hbm, tbl_v, sem_tbl)
    cm = pltpu.async_copy(mass_hbm, mass_v.at[pl.ds(0, _NUM_SPECIES)], sem_m)

    @pl.when(is_hi)
    def _():
        pltpu.async_copy(idx_hbm.at[pl.ds(row0, _K_HI * _CHUNK)],
                         idx_v.at[pl.ds(0, _K_HI * _CHUNK)], sem_i).wait()

    @pl.when(jnp.logical_not(is_hi))
    def _():
        pltpu.async_copy(idx_hbm.at[pl.ds(row0, _K_LO * _CHUNK)],
                         idx_v.at[pl.ds(0, _K_LO * _CHUNK)], sem_i).wait()

    ct.wait()
    cm.wait()

    def prow(r, carry):
        m = mass_v[pl.ds(r, 16)][0] * (1.0 / 90.0)
        for j in range(_DIM // 16):
            sl = pl.ds(j * 16, 16)
            tbl_v[r, sl] = tbl_v[r, sl] * m
        return carry

    lax.fori_loop(0, _NUM_SPECIES, prow, 0)

    def fill_half(stage, idx_off, g_lo, g_hi):
        @plsc.parallel_loop(g_lo, g_hi, unroll=4)
        def fgrp(g):
            idx16 = idx_v[pl.ds(idx_off + g * 16, 16)]
            rs = [idx16[i] for i in range(16)]
            for i in range(16):
                r = rs[i]
                vals = [tbl_v[r, pl.ds(j * 16, 16)]
                        for j in range(_DIM // 16)]
                for j in range(_DIM // 16):
                    stage[g * 16 + i, pl.ds(j * 16, 16)] = vals[j]

    def out_at(k, half):
        return out_hbm.at[pl.ds(row0 + k * _CHUNK + half * _HALF, _HALF)]

    def kbody(k, carry):
        @pl.when(k > 0)
        def _():
            pltpu.make_async_copy(stage_a, out_at(k - 1, 0), sem_oa).wait()
        fill_half(stage_a, k * _CHUNK, 0, 4)
        pltpu.async_copy(stage_a, out_at(k, 0), sem_oa)

        @pl.when(k > 0)
        def _():
            pltpu.make_async_copy(stage_b, out_at(k - 1, 1), sem_ob).wait()
        fill_half(stage_b, k * _CHUNK + _HALF, 0, 4)
        pltpu.async_copy(stage_b, out_at(k, 1), sem_ob)
        return carry

    lax.fori_loop(0, nk, kbody, 0)
    pltpu.make_async_copy(stage_a, out_at(nk - 1, 0), sem_oa).wait()
    pltpu.make_async_copy(stage_b, out_at(nk - 1, 1), sem_ob).wait()

    @pl.when(wid == _NW - 1)
    def _():
        t0 = _FULL * _CHUNK
        toff = _K_LO * _CHUNK
        pltpu.async_copy(idx_hbm.at[pl.ds(t0, _TAIL)],
                         idx_v.at[pl.ds(toff, _TAIL)], sem_i).wait()
        fill_half(stage_a, toff, 0, 2)
        pltpu.async_copy(
            stage_a.at[pl.ds(0, _TAIL)],
            out_hbm.at[pl.ds(t0, _TAIL)], sem_oa).wait()


def kernel(node_specie, embeddings, atomic_masses):
    return _sc_embed(embeddings, atomic_masses[:_NUM_SPECIES], node_specie)


# speedup vs baseline: 1.6688x; 1.6688x over previous
import functools

import jax
import jax.numpy as jnp
from jax import lax
from jax.experimental import pallas as pl
from jax.experimental.pallas import tpu as pltpu
from jax.experimental.pallas import tpu_sc as plsc

_N_NODES = 100000
_NUM_SPECIES = 119
_DIM = 128

_info = plsc.get_sparse_core_info()
_NC = _info.num_cores
_NS = _info.num_subcores
_NW = _NC * _NS

_CHUNK = 128
_HALF = _CHUNK // 2
_FULL = _N_NODES // _CHUNK
_TAIL = _N_NODES - _FULL * _CHUNK
_K_LO = _FULL // _NW
_N_HI = _FULL - _K_LO * _NW
_K_HI = _K_LO + 1

_mesh = plsc.VectorSubcoreMesh(core_axis_name="c", subcore_axis_name="s")


@functools.partial(
    pl.kernel,
    mesh=_mesh,
    out_type=jax.ShapeDtypeStruct((_N_NODES, _DIM), jnp.float32),
    scratch_types=[
        pltpu.VMEM((_NUM_SPECIES, _DIM), jnp.float32),
        pltpu.VMEM((_NUM_SPECIES + 17, ), jnp.float32),
        pltpu.VMEM((_K_HI * _CHUNK,), jnp.int32),
        pltpu.VMEM((_HALF, _DIM), jnp.float32),
        pltpu.VMEM((_HALF, _DIM), jnp.float32),
        pltpu.SemaphoreType.DMA,
        pltpu.SemaphoreType.DMA,
        pltpu.SemaphoreType.DMA,
        pltpu.SemaphoreType.DMA,
        pltpu.SemaphoreType.DMA,
    ],
)
def _sc_embed(table_hbm, mass_hbm, idx_hbm, out_hbm,
              tbl_v, mass_v, idx_v, stage_a, stage_b,
              sem_tbl, sem_m, sem_i, sem_oa, sem_ob):
    wid = lax.axis_index("s") * _NC + lax.axis_index("c")
    is_hi = wid < _N_HI
    base = lax.select(is_hi, wid * _K_HI,
                      _N_HI * _K_HI + (wid - _N_HI) * _K_LO)
    row0 = base * _CHUNK
    nk = lax.select(is_hi, _K_HI, _K_LO)

    ct = pltpu.async_copy(table_hbm, tbl_v, sem_tbl)
    cm = pltpu.async_copy(mass_hbm, mass_v.at[pl.ds(0, _NUM_SPECIES)], sem_m)

    @pl.when(is_hi)
    def _():
        pltpu.async_copy(idx_hbm.at[pl.ds(row0, _K_HI * _CHUNK)],
                         idx_v.at[pl.ds(0, _K_HI * _CHUNK)], sem_i)

    @pl.when(jnp.logical_not(is_hi))
    def _():
        pltpu.async_copy(idx_hbm.at[pl.ds(row0, _K_LO * _CHUNK)],
                         idx_v.at[pl.ds(0, _K_LO * _CHUNK)], sem_i)

    ct.wait()
    cm.wait()

    def prow(r, carry):
        m = mass_v[pl.ds(r, 16)][0] * (1.0 / 90.0)
        for j in range(_DIM // 16):
            sl = pl.ds(j * 16, 16)
            tbl_v[r, sl] = tbl_v[r, sl] * m
        return carry

    lax.fori_loop(0, _NUM_SPECIES, prow, 0)

    @pl.when(is_hi)
    def _():
        pltpu.make_async_copy(
            idx_hbm.at[pl.ds(row0, _K_HI * _CHUNK)],
            idx_v.at[pl.ds(0, _K_HI * _CHUNK)], sem_i).wait()

    @pl.when(jnp.logical_not(is_hi))
    def _():
        pltpu.make_async_copy(
            idx_hbm.at[pl.ds(row0, _K_LO * _CHUNK)],
            idx_v.at[pl.ds(0, _K_LO * _CHUNK)], sem_i).wait()

    def fill_half(stage, idx_off, g_lo, g_hi):
        @plsc.parallel_loop(g_lo, g_hi, unroll=2)
        def fgrp(g):
            idx16 = idx_v[pl.ds(idx_off + g * 16, 16)]
            rs = [idx16[i] for i in range(16)]
            for i in range(16):
                r = rs[i]
                vals = [tbl_v[r, pl.ds(j * 16, 16)]
                        for j in range(_DIM // 16)]
                for j in range(_DIM // 16):
                    stage[g * 16 + i, pl.ds(j * 16, 16)] = vals[j]

    def out_at(k, half):
        return out_hbm.at[pl.ds(row0 + k * _CHUNK + half * _HALF, _HALF)]

    def kbody(k, carry):
        @pl.when(k > 0)
        def _():
            pltpu.make_async_copy(stage_a, out_at(k - 1, 0), sem_oa).wait()
        fill_half(stage_a, k * _CHUNK, 0, 4)
        pltpu.async_copy(stage_a, out_at(k, 0), sem_oa)

        @pl.when(k > 0)
        def _():
            pltpu.make_async_copy(stage_b, out_at(k - 1, 1), sem_ob).wait()
        fill_half(stage_b, k * _CHUNK + _HALF, 0, 4)
        pltpu.async_copy(stage_b, out_at(k, 1), sem_ob)
        return carry

    lax.fori_loop(0, nk, kbody, 0)
    pltpu.make_async_copy(stage_a, out_at(nk - 1, 0), sem_oa).wait()
    pltpu.make_async_copy(stage_b, out_at(nk - 1, 1), sem_ob).wait()

    @pl.when(wid == _NW - 1)
    def _():
        t0 = _FULL * _CHUNK
        toff = _K_LO * _CHUNK
        pltpu.async_copy(idx_hbm.at[pl.ds(t0, _TAIL)],
                         idx_v.at[pl.ds(toff, _TAIL)], sem_i).wait()
        fill_half(stage_a, toff, 0, 2)
        pltpu.async_copy(
            stage_a.at[pl.ds(0, _TAIL)],
            out_hbm.at[pl.ds(t0, _TAIL)], sem_oa).wait()


def kernel(node_specie, embeddings, atomic_masses):
    return _sc_embed(embeddings, atomic_masses[:_NUM_SPECIES], node_specie)
